# Initial kernel scaffold; baseline (speedup 1.0000x reference)
import jax
import jax.numpy as jnp
from jax.experimental import pallas as pl


def kernel(x, Wr, W1, b1, W2, b2):
    raise NotImplementedError("write your pallas kernel here")


# speedup vs baseline: 2.0163x; 2.0163x over previous
import functools

import jax
import jax.numpy as jnp
from jax import lax
from jax.experimental import pallas as pl
from jax.experimental.pallas import tpu as pltpu
from jax.experimental.pallas import tpu_sc as plsc

D_MODEL = 768
D_FF = 3072
NUM_EXPERTS = 8
TOP_K = 2
N_TOKENS = 2048
N_SLOTS = N_TOKENS * TOP_K

BLK = 256
NB = N_SLOTS // BLK + NUM_EXPERTS
P_ROWS = NB * BLK

_SC_INFO = plsc.get_sparse_core_info()
NC = _SC_INFO.num_cores
NS = _SC_INFO.num_subcores
NW = NC * NS


def _router_kernel(x_ref, wr_ref, exp_ref, w_ref):
    logits = jnp.dot(x_ref[...], wr_ref[...], preferred_element_type=jnp.float32)
    n = logits.shape[0]
    io = lax.broadcasted_iota(jnp.int32, (n, NUM_EXPERTS), 1)
    m1 = jnp.max(logits, axis=1, keepdims=True)
    i1 = jnp.min(jnp.where(logits == m1, io, NUM_EXPERTS), axis=1, keepdims=True)
    masked = jnp.where(io == i1, -jnp.inf, logits)
    m2 = jnp.max(masked, axis=1, keepdims=True)
    i2 = jnp.min(jnp.where(masked == m2, io, NUM_EXPERTS), axis=1, keepdims=True)
    d = jnp.exp(m2 - m1)
    w1 = 1.0 / (1.0 + d)
    exp_ref[...] = jnp.concatenate([i1, i2], axis=1)
    w_ref[...] = jnp.concatenate([w1, 1.0 - w1], axis=1)


def _router(x_flat, Wr):
    return pl.pallas_call(
        _router_kernel,
        out_shape=(
            jax.ShapeDtypeStruct((N_TOKENS, TOP_K), jnp.int32),
            jax.ShapeDtypeStruct((N_TOKENS, TOP_K), jnp.float32),
        ),
    )(x_flat, Wr)


def _dispatch(experts, weights):
    e_flat = experts.reshape(-1)
    w_flat = weights.reshape(-1)
    order = jnp.argsort(e_flat, stable=True)
    sorted_e = e_flat[order]
    cnt = jnp.zeros((NUM_EXPERTS,), jnp.int32).at[e_flat].add(1)
    pc = ((cnt + BLK - 1) // BLK) * BLK
    poff = jnp.cumsum(pc) - pc
    cstart = jnp.cumsum(cnt) - cnt
    i = jnp.arange(N_SLOTS, dtype=jnp.int32)
    p_i = poff[sorted_e] + (i - cstart[sorted_e])
    row_token = jnp.zeros((P_ROWS,), jnp.int32).at[p_i].set(order // TOP_K)
    w_row = jnp.zeros((P_ROWS,), jnp.float32).at[p_i].set(w_flat[order])
    pos = jnp.zeros((N_SLOTS,), jnp.int32).at[order].set(p_i)
    bstart = jnp.arange(NB, dtype=jnp.int32) * BLK
    be = -jnp.ones((NB,), jnp.int32)
    for e in range(NUM_EXPERTS):
        be = jnp.where((bstart >= poff[e]) & (bstart < poff[e] + pc[e]), e, be)
    return row_token, w_row, pos, be


_G_CHUNK = 96
_G_PER_W = P_ROWS // NW


@functools.partial(
    pl.kernel,
    mesh=plsc.VectorSubcoreMesh(core_axis_name="c", subcore_axis_name="s"),
    out_type=jax.ShapeDtypeStruct((P_ROWS, D_MODEL), jnp.float32),
    scratch_types=[
        pltpu.VMEM((_G_CHUNK,), jnp.int32),
        pltpu.VMEM((_G_CHUNK, D_MODEL), jnp.float32),
        pltpu.SemaphoreType.DMA,
    ],
)
def _sc_gather(x_hbm, idx_hbm, xs_hbm, idx_v, rows_v, sem):
    wid = lax.axis_index("s") * NC + lax.axis_index("c")
    for c in range(_G_PER_W // _G_CHUNK):
        base = wid * _G_PER_W + c * _G_CHUNK
        pltpu.sync_copy(idx_hbm.at[pl.ds(base, _G_CHUNK)], idx_v)
        pltpu.async_copy(x_hbm.at[idx_v], rows_v, sem).wait()
        pltpu.sync_copy(rows_v, xs_hbm.at[pl.ds(base, _G_CHUNK)])


def _ffn_kernel(be_ref, xs_ref, w1_ref, b1_ref, w2_ref, b2_ref, wr_ref, out_ref):
    b = pl.program_id(0)

    @pl.when(be_ref[b] >= 0)
    def _():
        h = jnp.dot(xs_ref[...], w1_ref[0], preferred_element_type=jnp.float32)
        h = h + b1_ref[0]
        h = 0.5 * h * (1.0 + lax.erf(h * (2.0 ** -0.5)))
        y = jnp.dot(h, w2_ref[0], preferred_element_type=jnp.float32)
        out_ref[...] = (y + b2_ref[0]) * wr_ref[...]

    @pl.when(be_ref[b] < 0)
    def _():
        out_ref[...] = jnp.zeros_like(out_ref)


def _ffn(xs, W1, b1, W2, b2, w_row, be):
    def emap(b, be_ref):
        return (jnp.where(be_ref[b] < 0, NUM_EXPERTS - 1, be_ref[b]), 0, 0)

    def emap2(b, be_ref):
        return (jnp.where(be_ref[b] < 0, NUM_EXPERTS - 1, be_ref[b]), 0, 0)

    grid_spec = pltpu.PrefetchScalarGridSpec(
        num_scalar_prefetch=1,
        grid=(NB,),
        in_specs=[
            pl.BlockSpec((BLK, D_MODEL), lambda b, be_ref: (b, 0)),
            pl.BlockSpec((1, D_MODEL, D_FF), emap),
            pl.BlockSpec((1, 1, D_FF), emap2),
            pl.BlockSpec((1, D_FF, D_MODEL), emap),
            pl.BlockSpec((1, 1, D_MODEL), emap2),
            pl.BlockSpec((BLK, 1), lambda b, be_ref: (b, 0)),
        ],
        out_specs=pl.BlockSpec((BLK, D_MODEL), lambda b, be_ref: (b, 0)),
    )
    return pl.pallas_call(
        _ffn_kernel,
        grid_spec=grid_spec,
        out_shape=jax.ShapeDtypeStruct((P_ROWS, D_MODEL), jnp.float32),
    )(be, xs, W1, b1.reshape(NUM_EXPERTS, 1, D_FF), W2,
      b2.reshape(NUM_EXPERTS, 1, D_MODEL), w_row.reshape(P_ROWS, 1))


_C_TOK = N_TOKENS // NW
_C_LANES = D_MODEL // 16


@functools.partial(
    pl.kernel,
    mesh=plsc.VectorSubcoreMesh(core_axis_name="c", subcore_axis_name="s"),
    out_type=jax.ShapeDtypeStruct((N_TOKENS, D_MODEL), jnp.float32),
    scratch_types=[
        pltpu.VMEM((2 * _C_TOK,), jnp.int32),
        pltpu.VMEM((2 * _C_TOK, D_MODEL), jnp.float32),
        pltpu.SemaphoreType.DMA,
    ],
)
def _sc_combine(ys_hbm, pos_hbm, out_hbm, idx_v, buf_v, sem):
    wid = lax.axis_index("s") * NC + lax.axis_index("c")
    pltpu.sync_copy(pos_hbm.at[pl.ds(wid * 2 * _C_TOK, 2 * _C_TOK)], idx_v)
    pltpu.async_copy(ys_hbm.at[idx_v], buf_v, sem).wait()

    def body(i, _):
        for j in range(_C_LANES):
            s = pl.ds(j * 16, 16)
            buf_v[i, s] = buf_v[2 * i, s] + buf_v[2 * i + 1, s]
        return 0

    lax.fori_loop(0, _C_TOK, body, 0)
    pltpu.sync_copy(buf_v.at[pl.ds(0, _C_TOK)],
                    out_hbm.at[pl.ds(wid * _C_TOK, _C_TOK)])


def kernel(x, Wr, W1, b1, W2, b2):
    Bv, Tv, C = x.shape
    x_flat = x.reshape(-1, C)
    experts, weights = _router(x_flat, Wr)
    row_token, w_row, pos, be = _dispatch(experts, weights)
    xs = _sc_gather(x_flat, row_token)
    ys = _ffn(xs, W1, b1, W2, b2, w_row, be)
    out = _sc_combine(ys, pos)
    return out.reshape(Bv, Tv, C)


# speedup vs baseline: 2.0168x; 1.0003x over previous
import functools

import jax
import jax.numpy as jnp
from jax import lax
from jax.experimental import pallas as pl
from jax.experimental.pallas import tpu as pltpu
from jax.experimental.pallas import tpu_sc as plsc

D_MODEL = 768
D_FF = 3072
NUM_EXPERTS = 8
TOP_K = 2
N_TOKENS = 2048
N_SLOTS = N_TOKENS * TOP_K

BLK = 256
NB = N_SLOTS // BLK + NUM_EXPERTS
P_ROWS = NB * BLK

_SC_INFO = plsc.get_sparse_core_info()
NC = _SC_INFO.num_cores
NS = _SC_INFO.num_subcores
NW = NC * NS


def _router_kernel(x_ref, wr_ref, exp_ref, w_ref):
    logits = jnp.dot(x_ref[...], wr_ref[...], preferred_element_type=jnp.float32)
    n = logits.shape[0]
    io = lax.broadcasted_iota(jnp.int32, (n, NUM_EXPERTS), 1)
    m1 = jnp.max(logits, axis=1, keepdims=True)
    i1 = jnp.min(jnp.where(logits == m1, io, NUM_EXPERTS), axis=1, keepdims=True)
    masked = jnp.where(io == i1, -jnp.inf, logits)
    m2 = jnp.max(masked, axis=1, keepdims=True)
    i2 = jnp.min(jnp.where(masked == m2, io, NUM_EXPERTS), axis=1, keepdims=True)
    d = jnp.exp(m2 - m1)
    w1 = 1.0 / (1.0 + d)
    exp_ref[...] = jnp.concatenate([i1, i2], axis=1)
    w_ref[...] = jnp.concatenate([w1, 1.0 - w1], axis=1)


def _router(x_flat, Wr):
    return pl.pallas_call(
        _router_kernel,
        out_shape=(
            jax.ShapeDtypeStruct((N_TOKENS, TOP_K), jnp.int32),
            jax.ShapeDtypeStruct((N_TOKENS, TOP_K), jnp.float32),
        ),
    )(x_flat, Wr)


def _dispatch(experts, weights):
    e_flat = experts.reshape(-1)
    w_flat = weights.reshape(-1)
    order = jnp.argsort(e_flat, stable=True)
    sorted_e = e_flat[order]
    cnt = jnp.zeros((NUM_EXPERTS,), jnp.int32).at[e_flat].add(1)
    pc = ((cnt + BLK - 1) // BLK) * BLK
    poff = jnp.cumsum(pc) - pc
    cstart = jnp.cumsum(cnt) - cnt
    i = jnp.arange(N_SLOTS, dtype=jnp.int32)
    p_i = poff[sorted_e] + (i - cstart[sorted_e])
    row_token = jnp.zeros((P_ROWS,), jnp.int32).at[p_i].set(order // TOP_K)
    w_row = jnp.zeros((P_ROWS,), jnp.float32).at[p_i].set(w_flat[order])
    pos = jnp.zeros((N_SLOTS,), jnp.int32).at[order].set(p_i)
    bstart = jnp.arange(NB, dtype=jnp.int32) * BLK
    be = -jnp.ones((NB,), jnp.int32)
    for e in range(NUM_EXPERTS):
        be = jnp.where((bstart >= poff[e]) & (bstart < poff[e] + pc[e]), e, be)
    return row_token, w_row, pos, be


_G_PER_W = P_ROWS // NW
_G_CHUNK = 64
_G_NCH = _G_PER_W // _G_CHUNK


@functools.partial(
    pl.kernel,
    mesh=plsc.VectorSubcoreMesh(core_axis_name="c", subcore_axis_name="s"),
    out_type=jax.ShapeDtypeStruct((P_ROWS, D_MODEL), jnp.float32),
    scratch_types=[
        pltpu.VMEM((_G_PER_W,), jnp.int32),
        pltpu.VMEM((_G_CHUNK, D_MODEL), jnp.float32),
        pltpu.VMEM((_G_CHUNK, D_MODEL), jnp.float32),
        pltpu.SemaphoreType.DMA,
        pltpu.SemaphoreType.DMA,
    ],
)
def _sc_gather(x_hbm, idx_hbm, xs_hbm, idx_v, buf_a, buf_b, sem_a, sem_b):
    wid = lax.axis_index("s") * NC + lax.axis_index("c")
    base = wid * _G_PER_W
    pltpu.sync_copy(idx_hbm.at[pl.ds(base, _G_PER_W)], idx_v)
    bufs = (buf_a, buf_b)
    sems = (sem_a, sem_b)
    for c in range(2):
        pltpu.async_copy(
            x_hbm.at[idx_v.at[pl.ds(c * _G_CHUNK, _G_CHUNK)]], bufs[c], sems[c])
    for c in range(_G_NCH):
        pltpu.make_async_copy(
            x_hbm.at[idx_v.at[pl.ds(c * _G_CHUNK, _G_CHUNK)]],
            bufs[c % 2], sems[c % 2]).wait()
        pltpu.sync_copy(bufs[c % 2], xs_hbm.at[pl.ds(base + c * _G_CHUNK, _G_CHUNK)])
        if c + 2 < _G_NCH:
            pltpu.async_copy(
                x_hbm.at[idx_v.at[pl.ds((c + 2) * _G_CHUNK, _G_CHUNK)]],
                bufs[c % 2], sems[c % 2])


def _ffn_kernel(be_ref, xs_ref, w1_ref, b1_ref, w2_ref, b2_ref, wr_ref, out_ref):
    b = pl.program_id(0)

    @pl.when(be_ref[b] >= 0)
    def _():
        h = jnp.dot(xs_ref[...], w1_ref[0], preferred_element_type=jnp.float32)
        h = h + b1_ref[0]
        h = 0.5 * h * (1.0 + lax.erf(h * (2.0 ** -0.5)))
        y = jnp.dot(h, w2_ref[0], preferred_element_type=jnp.float32)
        out_ref[...] = (y + b2_ref[0]) * wr_ref[...]

    @pl.when(be_ref[b] < 0)
    def _():
        out_ref[...] = jnp.zeros_like(out_ref)


def _ffn(xs, W1, b1, W2, b2, w_row, be):
    def emap(b, be_ref):
        return (jnp.where(be_ref[b] < 0, NUM_EXPERTS - 1, be_ref[b]), 0, 0)

    def emap2(b, be_ref):
        return (jnp.where(be_ref[b] < 0, NUM_EXPERTS - 1, be_ref[b]), 0, 0)

    grid_spec = pltpu.PrefetchScalarGridSpec(
        num_scalar_prefetch=1,
        grid=(NB,),
        in_specs=[
            pl.BlockSpec((BLK, D_MODEL), lambda b, be_ref: (b, 0)),
            pl.BlockSpec((1, D_MODEL, D_FF), emap),
            pl.BlockSpec((1, 1, D_FF), emap2),
            pl.BlockSpec((1, D_FF, D_MODEL), emap),
            pl.BlockSpec((1, 1, D_MODEL), emap2),
            pl.BlockSpec((BLK, 1), lambda b, be_ref: (b, 0)),
        ],
        out_specs=pl.BlockSpec((BLK, D_MODEL), lambda b, be_ref: (b, 0)),
    )
    return pl.pallas_call(
        _ffn_kernel,
        grid_spec=grid_spec,
        out_shape=jax.ShapeDtypeStruct((P_ROWS, D_MODEL), jnp.float32),
    )(be, xs, W1, b1.reshape(NUM_EXPERTS, 1, D_FF), W2,
      b2.reshape(NUM_EXPERTS, 1, D_MODEL), w_row.reshape(P_ROWS, 1))


_C_TOK = N_TOKENS // NW
_C_LANES = D_MODEL // 16


@functools.partial(
    pl.kernel,
    mesh=plsc.VectorSubcoreMesh(core_axis_name="c", subcore_axis_name="s"),
    out_type=jax.ShapeDtypeStruct((N_TOKENS, D_MODEL), jnp.float32),
    scratch_types=[
        pltpu.VMEM((2 * _C_TOK,), jnp.int32),
        pltpu.VMEM((2 * _C_TOK, D_MODEL), jnp.float32),
        pltpu.SemaphoreType.DMA,
    ],
)
def _sc_combine(ys_hbm, pos_hbm, out_hbm, idx_v, buf_v, sem):
    wid = lax.axis_index("s") * NC + lax.axis_index("c")
    pltpu.sync_copy(pos_hbm.at[pl.ds(wid * 2 * _C_TOK, 2 * _C_TOK)], idx_v)
    pltpu.async_copy(ys_hbm.at[idx_v], buf_v, sem).wait()

    def body(i, _):
        for j in range(_C_LANES):
            s = pl.ds(j * 16, 16)
            buf_v[i, s] = buf_v[2 * i, s] + buf_v[2 * i + 1, s]
        return 0

    lax.fori_loop(0, _C_TOK, body, 0)
    pltpu.sync_copy(buf_v.at[pl.ds(0, _C_TOK)],
                    out_hbm.at[pl.ds(wid * _C_TOK, _C_TOK)])


def kernel(x, Wr, W1, b1, W2, b2):
    Bv, Tv, C = x.shape
    x_flat = x.reshape(-1, C)
    experts, weights = _router(x_flat, Wr)
    row_token, w_row, pos, be = _dispatch(experts, weights)
    xs = _sc_gather(x_flat, row_token)
    ys = _ffn(xs, W1, b1, W2, b2, w_row, be)
    out = _sc_combine(ys, pos)
    return out.reshape(Bv, Tv, C)
